# Initial kernel scaffold; baseline (speedup 1.0000x reference)
import jax
import jax.numpy as jnp
from jax.experimental import pallas as pl


def kernel(Xext, V, theta, w_int, w_ext, edge_index_int, edge_index_ext):
    raise NotImplementedError("write your pallas kernel here")


# speedup vs baseline: 8.6344x; 8.6344x over previous
import functools

import jax
import jax.numpy as jnp
from jax import lax
from jax.experimental import pallas as pl
from jax.experimental.pallas import tpu as pltpu
from jax.experimental.pallas import tpu_sc as plsc

N = 100000
N_PRE = 100000
B = 16
E_INT = 3200000
E_EXT = 1600000
ALPHA = 0.95
RHO = 0.99
BETA = 0.2
THETA0 = 1.0

NPAD = 102400
NC = 2
NS = 16
NW = NC * NS
K = 128
NROWS = 100096
ROWS_PER_TILE = NROWS // NS


def _stage_a_body(v_ref, th_ref, x_ref, vl_ref, tn_ref):
    v = v_ref[...]
    th = th_ref[...]
    x = (v >= th).astype(jnp.float32)
    x_ref[...] = x
    vl_ref[...] = ALPHA * v * (1.0 - x)
    tn_ref[...] = THETA0 + RHO * (th - THETA0) + BETA * x


def _stage_c_body(vl_ref, a0_ref, a1_ref, vn_ref):
    vn_ref[...] = vl_ref[...] + a0_ref[...] + a1_ref[...]


_GRID = 8
_BLK = NPAD // _GRID


def _ew_spec():
    return pl.BlockSpec((B, _BLK), lambda i: (0, i))


_stage_a = pl.pallas_call(
    _stage_a_body,
    grid=(_GRID,),
    in_specs=[_ew_spec(), _ew_spec()],
    out_specs=[_ew_spec(), _ew_spec(), _ew_spec()],
    out_shape=[jax.ShapeDtypeStruct((B, NPAD), jnp.float32)] * 3,
)

_stage_c = pl.pallas_call(
    _stage_c_body,
    grid=(_GRID,),
    in_specs=[_ew_spec(), _ew_spec(), _ew_spec()],
    out_specs=_ew_spec(),
    out_shape=jax.ShapeDtypeStruct((B, NPAD), jnp.float32),
)

def _sc_edges_body(xT, xextT, src_i, dst_i, w_i, src_e, dst_e, w_e, out,
                   sidx, didx, wbuf, rows, acc):
    c = lax.axis_index("c")
    s = lax.axis_index("s")
    wid = s * NC + c

    for k in range(K):
        rows[k] = jnp.zeros((16,), jnp.float32)
    zbase = s * ROWS_PER_TILE

    def zbody(i, carry):
        pltpu.sync_copy(rows, acc.at[pl.ds(zbase + i * K, K)])
        return carry

    nz = ROWS_PER_TILE // K
    ztail = ROWS_PER_TILE - nz * K
    lax.fori_loop(0, nz, zbody, 0)
    pltpu.sync_copy(rows.at[pl.ds(0, ztail)],
                    acc.at[pl.ds(zbase + nz * K, ztail)])
    plsc.subcore_barrier()

    def run_edges(tab, srcr, dstr, wr, n_chunks):
        n_iter = (n_chunks + NW - 1) // NW

        def body(i, carry):
            chunk = wid + i * NW

            @pl.when(chunk < n_chunks)
            def _():
                base = chunk * K
                pltpu.sync_copy(srcr.at[pl.ds(base, K)], sidx)
                pltpu.sync_copy(dstr.at[pl.ds(base, K)], didx)
                pltpu.sync_copy(wr.at[pl.ds(base, K)], wbuf)
                pltpu.sync_copy(tab.at[sidx], rows)
                for j in range(K // 16):
                    wv = wbuf[pl.ds(j * 16, 16)]
                    for l in range(16):
                        k = j * 16 + l
                        lane = jnp.full((16,), l, dtype=jnp.int32)
                        wsplat = wv.at[lane].get(mode="promise_in_bounds")
                        rows[k] = rows[k] * wsplat
                pltpu.sync_copy(rows, acc.at[didx], add=True)

            return carry

        lax.fori_loop(0, n_iter, body, 0)

    run_edges(xT, src_i, dst_i, w_i, E_INT // K)
    run_edges(xextT, src_e, dst_e, w_e, E_EXT // K)

    plsc.subcore_barrier()
    wb = s * ROWS_PER_TILE
    pltpu.sync_copy(acc.at[pl.ds(wb, ROWS_PER_TILE)],
                    out.at[pl.ds(c * NROWS + wb, ROWS_PER_TILE)])


_sc_edges = functools.partial(
    pl.kernel,
    out_type=jax.ShapeDtypeStruct((NC * NROWS, 16), jnp.float32),
    mesh=plsc.VectorSubcoreMesh(core_axis_name="c", subcore_axis_name="s"),
    scratch_types=[
        pltpu.VMEM((K,), jnp.int32),
        pltpu.VMEM((K,), jnp.int32),
        pltpu.VMEM((K,), jnp.float32),
        pltpu.VMEM((K, 16), jnp.float32),
        pltpu.VMEM_SHARED((NROWS, 16), jnp.float32),
    ],
    compiler_params=pltpu.CompilerParams(use_tc_tiling_on_sc=False),
)(_sc_edges_body)


def kernel(Xext, V, theta, w_int, w_ext, edge_index_int, edge_index_ext):
    pad = NPAD - N
    Vp = jnp.pad(V, ((0, 0), (0, pad)))
    thp = jnp.pad(theta, ((0, 0), (0, pad)))
    X_p, Vleak_p, theta_new_p = _stage_a(Vp, thp)
    X = X_p[:, :N]
    XT = X.T
    XextT = Xext.T

    acc = _sc_edges(XT, XextT,
                    edge_index_int[0], edge_index_int[1], w_int,
                    edge_index_ext[0], edge_index_ext[1], w_ext)
    a0T = jnp.pad(acc[:N].T, ((0, 0), (0, pad)))
    a1T = jnp.pad(acc[NROWS:NROWS + N].T, ((0, 0), (0, pad)))
    V_new_p = _stage_c(Vleak_p, a0T, a1T)
    return X, V_new_p[:, :N], theta_new_p[:, :N]


# speedup vs baseline: 19.4310x; 2.2504x over previous
import functools

import jax
import jax.numpy as jnp
from jax import lax
from jax.experimental import pallas as pl
from jax.experimental.pallas import tpu as pltpu
from jax.experimental.pallas import tpu_sc as plsc

N = 100000
N_PRE = 100000
B = 16
E_INT = 3200000
E_EXT = 1600000
ALPHA = 0.95
RHO = 0.99
BETA = 0.2
THETA0 = 1.0

NPAD = 102400
NC = 2
NS = 16
NW = NC * NS
K = 128
NROWS = 100096
ROWS_PER_TILE = NROWS // NS


def _stage_a_body(v_ref, th_ref, x_ref, vl_ref, tn_ref):
    v = v_ref[...]
    th = th_ref[...]
    x = (v >= th).astype(jnp.float32)
    x_ref[...] = x
    vl_ref[...] = ALPHA * v * (1.0 - x)
    tn_ref[...] = THETA0 + RHO * (th - THETA0) + BETA * x


def _stage_c_body(vl_ref, a0_ref, a1_ref, vn_ref):
    vn_ref[...] = vl_ref[...] + a0_ref[...] + a1_ref[...]


_GRID = 8
_BLK = NPAD // _GRID


def _ew_spec():
    return pl.BlockSpec((B, _BLK), lambda i: (0, i))


_stage_a = pl.pallas_call(
    _stage_a_body,
    grid=(_GRID,),
    in_specs=[_ew_spec(), _ew_spec()],
    out_specs=[_ew_spec(), _ew_spec(), _ew_spec()],
    out_shape=[jax.ShapeDtypeStruct((B, NPAD), jnp.float32)] * 3,
)

_stage_c = pl.pallas_call(
    _stage_c_body,
    grid=(_GRID,),
    in_specs=[_ew_spec(), _ew_spec(), _ew_spec()],
    out_specs=_ew_spec(),
    out_shape=jax.ShapeDtypeStruct((B, NPAD), jnp.float32),
)

def _sc_edges_body(xT, xextT, src_i, dst_i, w_i, src_e, dst_e, w_e, out,
                   sblk, dblk, wblk, rowsA, rowsB, acc,
                   sg0, sg1, ss0, ss1):
    c = lax.axis_index("c")
    s = lax.axis_index("s")
    wid = s * NC + c
    rows = [rowsA, rowsB]
    sem_g = [sg0, sg1]
    sem_s = [ss0, ss1]

    for k in range(K):
        rowsA[k] = jnp.zeros((16,), jnp.float32)
    zbase = s * ROWS_PER_TILE
    nz = ROWS_PER_TILE // K
    ztail = ROWS_PER_TILE - nz * K
    zd = [pltpu.async_copy(rowsA, acc.at[pl.ds(zbase + i * K, K)], sg0)
          for i in range(nz)]
    zd.append(pltpu.async_copy(rowsA.at[pl.ds(0, ztail)],
                               acc.at[pl.ds(zbase + nz * K, ztail)], sg0))
    for d in zd:
        d.wait()
    plsc.subcore_barrier()

    def run_edges(tab, src2, dst2, w2, n_chunks, cb):
        n_blocks = n_chunks // cb
        n_my = (n_blocks - 1 - wid) // NW + 1

        def body(i, carry):
            blk = wid + i * NW
            base = blk * cb
            pltpu.sync_copy(src2.at[pl.ds(base, cb)], sblk.at[pl.ds(0, cb)])
            pltpu.sync_copy(dst2.at[pl.ds(base, cb)], dblk.at[pl.ds(0, cb)])
            pltpu.sync_copy(w2.at[pl.ds(base, cb)], wblk.at[pl.ds(0, cb)])
            gd = {0: pltpu.async_copy(tab.at[sblk.at[0]], rows[0], sem_g[0])}
            sd = {}
            for t in range(cb):
                b = t % 2
                if t >= 1:
                    sd[t - 1].wait()
                if t + 1 < cb:
                    nb = (t + 1) % 2
                    gd[t + 1] = pltpu.async_copy(tab.at[sblk.at[t + 1]],
                                                 rows[nb], sem_g[nb])
                gd[t].wait()
                for j in range(K // 16):
                    wv = wblk[t, pl.ds(j * 16, 16)]
                    for l in range(16):
                        k = j * 16 + l
                        lane = jnp.full((16,), l, dtype=jnp.int32)
                        wsplat = wv.at[lane].get(mode="promise_in_bounds")
                        rows[b][k] = rows[b][k] * wsplat
                sd[t] = pltpu.async_copy(rows[b], acc.at[dblk.at[t]],
                                         sem_s[b], add=True)
            sd[cb - 1].wait()
            return carry

        lax.fori_loop(0, n_my, body, 0)

    run_edges(xT, src_i, dst_i, w_i, E_INT // K, 8)
    run_edges(xextT, src_e, dst_e, w_e, E_EXT // K, 4)

    plsc.subcore_barrier()
    wb = s * ROWS_PER_TILE
    pltpu.sync_copy(acc.at[pl.ds(wb, ROWS_PER_TILE)],
                    out.at[pl.ds(c * NROWS + wb, ROWS_PER_TILE)])


_sc_edges = functools.partial(
    pl.kernel,
    out_type=jax.ShapeDtypeStruct((NC * NROWS, 16), jnp.float32),
    mesh=plsc.VectorSubcoreMesh(core_axis_name="c", subcore_axis_name="s"),
    scratch_types=[
        pltpu.VMEM((8, K), jnp.int32),
        pltpu.VMEM((8, K), jnp.int32),
        pltpu.VMEM((8, K), jnp.float32),
        pltpu.VMEM((K, 16), jnp.float32),
        pltpu.VMEM((K, 16), jnp.float32),
        pltpu.VMEM_SHARED((NROWS, 16), jnp.float32),
        pltpu.SemaphoreType.DMA,
        pltpu.SemaphoreType.DMA,
        pltpu.SemaphoreType.DMA,
        pltpu.SemaphoreType.DMA,
    ],
    compiler_params=pltpu.CompilerParams(use_tc_tiling_on_sc=False),
)(_sc_edges_body)


def kernel(Xext, V, theta, w_int, w_ext, edge_index_int, edge_index_ext):
    pad = NPAD - N
    Vp = jnp.pad(V, ((0, 0), (0, pad)))
    thp = jnp.pad(theta, ((0, 0), (0, pad)))
    X_p, Vleak_p, theta_new_p = _stage_a(Vp, thp)
    X = X_p[:, :N]
    XT = X.T
    XextT = Xext.T

    acc = _sc_edges(XT, XextT,
                    edge_index_int[0].reshape(E_INT // K, K),
                    edge_index_int[1].reshape(E_INT // K, K),
                    w_int.reshape(E_INT // K, K),
                    edge_index_ext[0].reshape(E_EXT // K, K),
                    edge_index_ext[1].reshape(E_EXT // K, K),
                    w_ext.reshape(E_EXT // K, K))
    a0T = jnp.pad(acc[:N].T, ((0, 0), (0, pad)))
    a1T = jnp.pad(acc[NROWS:NROWS + N].T, ((0, 0), (0, pad)))
    V_new_p = _stage_c(Vleak_p, a0T, a1T)
    return X, V_new_p[:, :N], theta_new_p[:, :N]


# speedup vs baseline: 22.6521x; 1.1658x over previous
import functools

import jax
import jax.numpy as jnp
from jax import lax
from jax.experimental import pallas as pl
from jax.experimental.pallas import tpu as pltpu
from jax.experimental.pallas import tpu_sc as plsc

N = 100000
N_PRE = 100000
B = 16
E_INT = 3200000
E_EXT = 1600000
ALPHA = 0.95
RHO = 0.99
BETA = 0.2
THETA0 = 1.0

NC = 2
NS = 16
NW = NC * NS
K = 128
RB = 4
NROWS = 100096
ROWS_PER_TILE = NROWS // NS


def _stage_a_body(v_ref, th_ref, x_ref, vl_ref, tn_ref):
    v = v_ref[...]
    th = th_ref[...]
    x = (v >= th).astype(jnp.float32)
    x_ref[...] = x
    vl_ref[...] = ALPHA * v * (1.0 - x)
    tn_ref[...] = THETA0 + RHO * (th - THETA0) + BETA * x


def _stage_c_body(vl_ref, cur_ref, vn_ref):
    vn_ref[...] = vl_ref[...] + cur_ref[...]


_stage_a = pl.pallas_call(
    _stage_a_body,
    out_shape=[jax.ShapeDtypeStruct((B, N), jnp.float32)] * 3,
)

_stage_c = pl.pallas_call(
    _stage_c_body,
    out_shape=jax.ShapeDtypeStruct((B, N), jnp.float32),
)


def _sc_edges_body(xT, xextT, ei, we_i, ee, we_e, out,
                   sblk, dblk, wblk, rows, acc, *sems):
    c = lax.axis_index("c")
    s = lax.axis_index("s")
    wid = s * NC + c
    sem_g = sems[:RB]
    sem_s = sems[RB:]

    for k in range(K):
        rows[0, k] = jnp.zeros((16,), jnp.float32)
    zbase = s * ROWS_PER_TILE
    nz = ROWS_PER_TILE // K
    ztail = ROWS_PER_TILE - nz * K
    zd = [pltpu.async_copy(rows.at[0], acc.at[pl.ds(zbase + i * K, K)], sem_g[0])
          for i in range(nz)]
    zd.append(pltpu.async_copy(rows.at[0, pl.ds(0, ztail)],
                               acc.at[pl.ds(zbase + nz * K, ztail)], sem_g[0]))
    for d in zd:
        d.wait()
    plsc.subcore_barrier()

    def run_edges(tab, src2, dst2, w2, n_chunks, cb):
        n_blocks = n_chunks // cb
        n_my = (n_blocks - 1 - wid) // NW + 1

        def body(i, carry):
            blk = wid + i * NW
            base = blk * cb
            pltpu.sync_copy(src2.at[pl.ds(base, cb)], sblk.at[pl.ds(0, cb)])
            pltpu.sync_copy(dst2.at[pl.ds(base, cb)], dblk.at[pl.ds(0, cb)])
            pltpu.sync_copy(w2.at[pl.ds(base, cb)], wblk.at[pl.ds(0, cb)])
            gd = {}
            sd = {}

            def gather(t):
                b = t % RB
                gd[t] = pltpu.async_copy(tab.at[sblk.at[t]], rows.at[b],
                                         sem_g[b])

            gather(0)
            gather(1)
            pending = []
            for t in range(cb):
                b = t % RB
                gd[t].wait()
                for j in range(K // 16):
                    wv = wblk[t, pl.ds(j * 16, 16)]
                    for l in range(16):
                        k = j * 16 + l
                        lane = jnp.full((16,), l, dtype=jnp.int32)
                        wsplat = wv.at[lane].get(mode="promise_in_bounds")
                        rows[b, k] = rows[b, k] * wsplat
                sd[t] = pltpu.async_copy(rows.at[b], acc.at[dblk.at[t]],
                                         sem_s[b], add=True)
                pending.append(t)
                nxt = t + 2
                if nxt < cb:
                    if nxt - RB in pending:
                        sd[nxt - RB].wait()
                        pending.remove(nxt - RB)
                    gather(nxt)
            for t in pending:
                sd[t].wait()
            return carry

        lax.fori_loop(0, n_my, body, 0)

    run_edges(xT, ei.at[0], ei.at[1], we_i, E_INT // K, 8)
    run_edges(xextT, ee.at[0], ee.at[1], we_e, E_EXT // K, 4)

    plsc.subcore_barrier()
    wb = s * ROWS_PER_TILE
    pltpu.sync_copy(acc.at[pl.ds(wb, ROWS_PER_TILE)],
                    out.at[pl.ds(c * NROWS + wb, ROWS_PER_TILE)])


_sc_edges = functools.partial(
    pl.kernel,
    out_type=jax.ShapeDtypeStruct((NC * NROWS, 16), jnp.float32),
    mesh=plsc.VectorSubcoreMesh(core_axis_name="c", subcore_axis_name="s"),
    scratch_types=[
        pltpu.VMEM((8, K), jnp.int32),
        pltpu.VMEM((8, K), jnp.int32),
        pltpu.VMEM((8, K), jnp.float32),
        pltpu.VMEM((RB, K, 16), jnp.float32),
        pltpu.VMEM_SHARED((NROWS, 16), jnp.float32),
    ] + [pltpu.SemaphoreType.DMA] * (2 * RB),
    compiler_params=pltpu.CompilerParams(use_tc_tiling_on_sc=False),
)(_sc_edges_body)


def kernel(Xext, V, theta, w_int, w_ext, edge_index_int, edge_index_ext):
    X, Vleak, theta_new = _stage_a(V, theta)
    XT = X.T
    XextT = Xext.T

    acc = _sc_edges(XT, XextT,
                    edge_index_int.reshape(2, E_INT // K, K),
                    w_int.reshape(E_INT // K, K),
                    edge_index_ext.reshape(2, E_EXT // K, K),
                    w_ext.reshape(E_EXT // K, K))
    curT = (acc[:N] + acc[NROWS:NROWS + N]).T
    V_new = _stage_c(Vleak, curT)
    return X, V_new, theta_new


# speedup vs baseline: 29.0987x; 1.2846x over previous
import functools

import jax
import jax.numpy as jnp
from jax import lax
from jax.experimental import pallas as pl
from jax.experimental.pallas import tpu as pltpu
from jax.experimental.pallas import tpu_sc as plsc

N = 100000
N_PRE = 100000
B = 16
E_INT = 3200000
E_EXT = 1600000
ALPHA = 0.95
RHO = 0.99
BETA = 0.2
THETA0 = 1.0

NC = 2
NS = 16
NW = NC * NS
K = 128
RB = 4
NROWS = 100096
ROWS_PER_TILE = NROWS // NS


def _stage_a_body(v_ref, th_ref, x_ref, vl_ref, tn_ref):
    v = v_ref[...]
    th = th_ref[...]
    x = (v >= th).astype(jnp.float32)
    x_ref[...] = x
    vl_ref[...] = ALPHA * v * (1.0 - x)
    tn_ref[...] = THETA0 + RHO * (th - THETA0) + BETA * x


def _stage_c_body(vl_ref, cur_ref, vn_ref):
    vn_ref[...] = vl_ref[...] + cur_ref[...]


_stage_a = pl.pallas_call(
    _stage_a_body,
    out_shape=[jax.ShapeDtypeStruct((B, N), jnp.float32)] * 3,
)

_stage_c = pl.pallas_call(
    _stage_c_body,
    out_shape=jax.ShapeDtypeStruct((B, N), jnp.float32),
)


def _sc_edges_body(xT, xextT, ei, we_i, ee, we_e, out,
                   sblk, dblk, wblk, rows, acc, *sems):
    c = lax.axis_index("c")
    s = lax.axis_index("s")
    wid = s * NC + c
    sem_g = sems[:RB]
    sem_s = sems[RB:2 * RB]
    sem_i = sems[2 * RB:]

    for k in range(K):
        rows[0, k] = jnp.zeros((16,), jnp.float32)
    zbase = s * ROWS_PER_TILE
    nz = ROWS_PER_TILE // K
    ztail = ROWS_PER_TILE - nz * K
    zd = [pltpu.async_copy(rows.at[0], acc.at[pl.ds(zbase + i * K, K)], sem_g[0])
          for i in range(nz)]
    zd.append(pltpu.async_copy(rows.at[0, pl.ds(0, ztail)],
                               acc.at[pl.ds(zbase + nz * K, ztail)], sem_g[0]))
    for d in zd:
        d.wait()
    plsc.subcore_barrier()

    def run_edges(tab, src2, dst2, w2, n_chunks, cb):
        n_blocks = n_chunks // cb
        n_my = (n_blocks - 1 - wid) // NW + 1

        def issue(iblk, buf, sem):
            blk = jnp.minimum(wid + iblk * NW, n_blocks - 1)
            base = blk * cb
            pltpu.async_copy(src2.at[pl.ds(base, cb)],
                             sblk.at[buf, pl.ds(0, cb)], sem)
            pltpu.async_copy(dst2.at[pl.ds(base, cb)],
                             dblk.at[buf, pl.ds(0, cb)], sem)
            pltpu.async_copy(w2.at[pl.ds(base, cb)],
                             wblk.at[buf, pl.ds(0, cb)], sem)

        def wait3(buf, sem):
            pltpu.make_async_copy(src2.at[pl.ds(0, cb)],
                                  sblk.at[buf, pl.ds(0, cb)], sem).wait()
            pltpu.make_async_copy(dst2.at[pl.ds(0, cb)],
                                  dblk.at[buf, pl.ds(0, cb)], sem).wait()
            pltpu.make_async_copy(w2.at[pl.ds(0, cb)],
                                  wblk.at[buf, pl.ds(0, cb)], sem).wait()

        def compute(buf):
            gd = {}
            sd = {}

            def gather(t):
                b = t % RB
                gd[t] = pltpu.async_copy(tab.at[sblk.at[buf, t]], rows.at[b],
                                         sem_g[b])

            gather(0)
            gather(1)
            pending = []
            for t in range(cb):
                b = t % RB
                gd[t].wait()
                for j in range(K // 16):
                    wv = wblk[buf, t, pl.ds(j * 16, 16)]
                    for l in range(16):
                        k = j * 16 + l
                        lane = jnp.full((16,), l, dtype=jnp.int32)
                        wsplat = wv.at[lane].get(mode="promise_in_bounds")
                        rows[b, k] = rows[b, k] * wsplat
                sd[t] = pltpu.async_copy(rows.at[b], acc.at[dblk.at[buf, t]],
                                         sem_s[b], add=True)
                pending.append(t)
                nxt = t + 2
                if nxt < cb:
                    if nxt - RB in pending:
                        sd[nxt - RB].wait()
                        pending.remove(nxt - RB)
                    gather(nxt)
            for t in pending:
                sd[t].wait()

        issue(0, 0, sem_i[0])

        def body(p, carry):
            wait3(0, sem_i[0])
            issue(2 * p + 1, 1, sem_i[1])
            compute(0)
            wait3(1, sem_i[1])
            issue(2 * p + 2, 0, sem_i[0])
            compute(1)
            return carry

        lax.fori_loop(0, n_my // 2, body, 0)
        wait3(0, sem_i[0])

        @pl.when(n_my % 2 == 1)
        def _tail():
            compute(0)

    run_edges(xT, ei.at[0], ei.at[1], we_i, E_INT // K, 8)
    run_edges(xextT, ee.at[0], ee.at[1], we_e, E_EXT // K, 4)

    plsc.subcore_barrier()
    wb = s * ROWS_PER_TILE
    pltpu.sync_copy(acc.at[pl.ds(wb, ROWS_PER_TILE)],
                    out.at[pl.ds(c * NROWS + wb, ROWS_PER_TILE)])


_sc_edges = functools.partial(
    pl.kernel,
    out_type=jax.ShapeDtypeStruct((NC * NROWS, 16), jnp.float32),
    mesh=plsc.VectorSubcoreMesh(core_axis_name="c", subcore_axis_name="s"),
    scratch_types=[
        pltpu.VMEM((2, 8, K), jnp.int32),
        pltpu.VMEM((2, 8, K), jnp.int32),
        pltpu.VMEM((2, 8, K), jnp.float32),
        pltpu.VMEM((RB, K, 16), jnp.float32),
        pltpu.VMEM_SHARED((NROWS, 16), jnp.float32),
    ] + [pltpu.SemaphoreType.DMA] * (2 * RB + 2),
    compiler_params=pltpu.CompilerParams(use_tc_tiling_on_sc=False),
)(_sc_edges_body)


def kernel(Xext, V, theta, w_int, w_ext, edge_index_int, edge_index_ext):
    X, Vleak, theta_new = _stage_a(V, theta)
    XT = X.T
    XextT = Xext.T

    acc = _sc_edges(XT, XextT,
                    edge_index_int.reshape(2, E_INT // K, K),
                    w_int.reshape(E_INT // K, K),
                    edge_index_ext.reshape(2, E_EXT // K, K),
                    w_ext.reshape(E_EXT // K, K))
    curT = (acc[:N] + acc[NROWS:NROWS + N]).T
    V_new = _stage_c(Vleak, curT)
    return X, V_new, theta_new
